# baseline (speedup 1.0000x reference)
import functools

import jax
import jax.numpy as jnp
from jax import lax
from jax.experimental import pallas as pl
from jax.experimental.pallas import tpu as pltpu
from jax.experimental.pallas import tpu_sc as plsc

_NUM_CORES = 2
_NUM_SUBCORES = 16
_NW = _NUM_CORES * _NUM_SUBCORES
_L = 16


def _embed_t(xt, tab2):
    s_len, b_len = xt.shape
    d = tab2.shape[1]
    w_b = b_len // _NW
    dg8 = d // 8
    bt = w_b // 2 // 128
    n_j = w_b // 2 // _L

    mesh = plsc.VectorSubcoreMesh(
        core_axis_name="c", subcore_axis_name="s",
        num_cores=_NUM_CORES, num_subcores=_NUM_SUBCORES)

    @functools.partial(
        pl.kernel,
        mesh=mesh,
        compiler_params=pltpu.CompilerParams(
            use_tc_tiling_on_sc=False, needs_layout_passes=False),
        out_type=jax.ShapeDtypeStruct(
            (s_len, dg8, b_len // 128, 8, 128), jnp.float32),
        scratch_types=[
            pltpu.VMEM((s_len, w_b), jnp.int32),
            pltpu.VMEM((2, w_b), jnp.int32),
            pltpu.VMEM((2, w_b, d), jnp.float32),
            pltpu.VMEM((2, 1, dg8, bt, 8, 128), jnp.float32),
            pltpu.SemaphoreType.DMA,
            pltpu.SemaphoreType.DMA,
            pltpu.SemaphoreType.DMA,
            pltpu.SemaphoreType.DMA,
        ],
    )
    def emb(tab_hbm, xt_hbm, out_hbm, x_v, idx_v, buf, buf_t,
            gsem0, gsem1, osem0, osem1):
        wid = lax.axis_index("s") * _NUM_CORES + lax.axis_index("c")
        b0 = wid * w_b
        bt0 = b0 // 128
        gsems = (gsem0, gsem1)
        osems = (osem0, osem1)
        iot = lax.iota(jnp.int32, _L)

        pltpu.sync_copy(xt_hbm.at[:, pl.ds(b0, w_b)], x_v)

        def start_gather(s, slot):
            @plsc.parallel_loop(0, w_b // _L)
            def _prep(j):
                idx_v[slot, pl.ds(j * _L, _L)] = (
                    x_v[s, pl.ds(j * _L, _L)] << 1)

            pltpu.async_copy(
                tab_hbm.at[idx_v.at[slot]], buf.at[slot], gsems[slot])

        def wait_gather(slot):
            pltpu.make_async_copy(
                tab_hbm.at[idx_v.at[slot]], buf.at[slot],
                gsems[slot]).wait()

        def out_dma(s, h, slot):
            return pltpu.make_async_copy(
                buf_t.at[slot],
                out_hbm.at[pl.ds(s, 1), :, pl.ds(bt0 + h * bt, bt)],
                osems[slot])

        start_gather(0, 0)

        def body2(q, _):
            for slot in range(2):
                s = q * 2 + slot

                @pl.when(s + 1 < s_len)
                def _start_next():
                    start_gather(s + 1, 1 - slot)

                wait_gather(slot)

                for h in range(2):
                    @pl.when(s >= 1)
                    def _drain_out():
                        out_dma(s, h, h).wait()

                    for t in range(2):
                        base = h * (w_b // 2) + t * 128
                        tvec = iot * 0 + t

                        @plsc.parallel_loop(0, 8 * d, unroll=2)
                        def tbody(i):
                            r0 = i >> 6
                            c = i & (d - 1)
                            rloc = r0 * _L + iot
                            cvec = (c + iot) & (d - 1)
                            v = plsc.load_gather(
                                buf.at[slot], [base + rloc, cvec])
                            plsc.store_scatter(
                                buf_t.at[h, 0],
                                [cvec >> 3, tvec, cvec & 7, rloc], v)

                    out_dma(s, h, h).start()
            return 0

        lax.fori_loop(0, s_len // 2, body2, 0)

        for h in range(2):
            out_dma(s_len - 1, h, h).wait()

    return emb(tab2, xt)


def kernel(x, table):
    b, s = x.shape
    v, d = table.shape
    xt = x.T.astype(jnp.int32)
    tab2 = jnp.concatenate([table, table], axis=1).reshape(2 * v, d)
    out5 = _embed_t(xt, tab2)
    out = out5.transpose(2, 4, 0, 1, 3).reshape(b, s, d)
    return out


# speedup vs baseline: 1.1942x; 1.1942x over previous
import functools

import jax
import jax.numpy as jnp
from jax import lax
from jax.experimental import pallas as pl
from jax.experimental.pallas import tpu as pltpu
from jax.experimental.pallas import tpu_sc as plsc

_NUM_CORES = 2
_NUM_SUBCORES = 16
_NW = _NUM_CORES * _NUM_SUBCORES
_L = 16


def _embed_t(xt, tab2):
    s_len, b_len = xt.shape
    d = tab2.shape[1]
    w_b = b_len // _NW
    dg8 = d // 8
    bt = w_b // 2 // 128
    n_j = w_b // 2 // _L

    mesh = plsc.VectorSubcoreMesh(
        core_axis_name="c", subcore_axis_name="s",
        num_cores=_NUM_CORES, num_subcores=_NUM_SUBCORES)

    @functools.partial(
        pl.kernel,
        mesh=mesh,
        compiler_params=pltpu.CompilerParams(
            use_tc_tiling_on_sc=False, needs_layout_passes=False),
        out_type=jax.ShapeDtypeStruct(
            (s_len, dg8, b_len // 128, 8, 128), jnp.float32),
        scratch_types=[
            pltpu.VMEM((s_len, w_b), jnp.int32),
            pltpu.VMEM((2, w_b), jnp.int32),
            pltpu.VMEM((2, w_b, d), jnp.float32),
            pltpu.VMEM((2, 1, dg8, bt, 8, 128), jnp.float32),
            pltpu.SemaphoreType.DMA,
            pltpu.SemaphoreType.DMA,
            pltpu.SemaphoreType.DMA,
            pltpu.SemaphoreType.DMA,
        ],
    )
    def emb(tab_hbm, xt_hbm, out_hbm, x_v, idx_v, buf, buf_t,
            gsem0, gsem1, osem0, osem1):
        wid = lax.axis_index("s") * _NUM_CORES + lax.axis_index("c")
        b0 = wid * w_b
        bt0 = b0 // 128
        gsems = (gsem0, gsem1)
        osems = (osem0, osem1)
        iot = lax.iota(jnp.int32, _L)

        pltpu.sync_copy(xt_hbm.at[:, pl.ds(b0, w_b)], x_v)

        def start_gather(s, slot):
            @plsc.parallel_loop(0, w_b // _L)
            def _prep(j):
                idx_v[slot, pl.ds(j * _L, _L)] = (
                    x_v[s, pl.ds(j * _L, _L)] << 1)

            pltpu.async_copy(
                tab_hbm.at[idx_v.at[slot]], buf.at[slot], gsems[slot])

        def wait_gather(slot):
            pltpu.make_async_copy(
                tab_hbm.at[idx_v.at[slot]], buf.at[slot],
                gsems[slot]).wait()

        def out_dma(s, h, slot):
            return pltpu.make_async_copy(
                buf_t.at[slot],
                out_hbm.at[pl.ds(s, 1), :, pl.ds(bt0 + h * bt, bt)],
                osems[slot])

        start_gather(0, 0)

        def body2(q, _):
            for slot in range(2):
                s = q * 2 + slot

                @pl.when(s + 1 < s_len)
                def _start_next():
                    start_gather(s + 1, 1 - slot)

                wait_gather(slot)

                for h in range(2):
                    @pl.when(s >= 1)
                    def _drain_out():
                        out_dma(s, h, h).wait()

                    for t in range(2):
                        base = h * (w_b // 2) + t * 128
                        tvec = iot * 0 + t

                        @plsc.parallel_loop(0, 8 * d, unroll=2)
                        def tbody(i):
                            r0 = i >> 6
                            c = i & (d - 1)
                            rloc = r0 * _L + iot
                            cvec = (c + iot) & (d - 1)
                            v = plsc.load_gather(
                                buf.at[slot], [base + rloc, cvec])
                            plsc.store_scatter(
                                buf_t.at[h, 0],
                                [cvec >> 3, tvec, cvec & 7, rloc], v)

                    out_dma(s, h, h).start()
            return 0

        lax.fori_loop(0, s_len // 2, body2, 0)

        for h in range(2):
            out_dma(s_len - 1, h, h).wait()

    return emb(tab2, xt)


def kernel(x, table):
    b, s = x.shape
    v, d = table.shape
    xt = x.T.astype(jnp.int32)
    tab2 = jnp.pad(table, ((0, 0), (0, d))).reshape(2 * v, d)
    out5 = _embed_t(xt, tab2)
    out = out5.transpose(2, 4, 0, 1, 3).reshape(b, s, d)
    return out
